# Initial kernel scaffold; baseline (speedup 1.0000x reference)
import jax
import jax.numpy as jnp
from jax.experimental import pallas as pl


def kernel(atom_features, edge_index, edge_attr, pair_indices, pair_features, W_emb, b_emb, We1, be1, We2, be2, roots, root_bias, gammas, betas, Wp1, bp1, Wp2, bp2, Wp3, bp3):
    raise NotImplementedError("write your pallas kernel here")


# speedup vs baseline: 1.0077x; 1.0077x over previous
import functools

import jax
import jax.numpy as jnp
from jax import lax
from jax.experimental import pallas as pl
from jax.experimental.pallas import tpu as pltpu
from jax.experimental.pallas import tpu_sc as plsc

N = 10000
E = 160000
P = 50000
FA = 128
FP = 16
H = 32

NC = 2
NS = 16
NW = NC * NS

ECH = 5
CHUNK = 1024
E_PAD = NW * ECH * CHUNK
NPAD = 10240
ROWS_PER_TILE = NPAD // NS

PCH = 5
PROWS = 640
P_FLAT = NW * PCH * PROWS

MBLK = 1024
BLK_P = 2000


def _embed_body(atom_ref, wemb_ref, bemb_ref, x0_ref):
    x0_ref[...] = (
        jnp.dot(atom_ref[...], wemb_ref[...], preferred_element_type=jnp.float32)
        + bemb_ref[...]
    )


_embed_call = pl.pallas_call(
    _embed_body,
    out_shape=jax.ShapeDtypeStruct((N, H), jnp.float32),
)


def _msg_body(a_ref, xs_ref, w1_ref, b1_ref, w2_ref, b2_ref, o_ref):
    h = jnp.maximum(a_ref[...] * w1_ref[...] + b1_ref[...], 0.0)
    we = jnp.dot(h, w2_ref[...], preferred_element_type=jnp.float32) + b2_ref[...]
    we_b = we.astype(jnp.bfloat16).astype(jnp.float32)
    xs_b = xs_ref[...].astype(jnp.bfloat16).astype(jnp.float32)
    acc = jnp.zeros((MBLK, H), jnp.float32)
    for hh in range(H):
        acc = acc + xs_b[:, hh:hh + 1] * we_b[:, hh * H:(hh + 1) * H]
    o_ref[...] = acc


_msg_call = pl.pallas_call(
    _msg_body,
    grid=(E_PAD // MBLK,),
    in_specs=[
        pl.BlockSpec((MBLK, 1), lambda i: (i, 0)),
        pl.BlockSpec((MBLK, H), lambda i: (i, 0)),
        pl.BlockSpec((1, H), lambda i: (0, 0)),
        pl.BlockSpec((1, H), lambda i: (0, 0)),
        pl.BlockSpec((H, H * H), lambda i: (0, 0)),
        pl.BlockSpec((1, H * H), lambda i: (0, 0)),
    ],
    out_specs=pl.BlockSpec((MBLK, H), lambda i: (i, 0)),
    out_shape=jax.ShapeDtypeStruct((E_PAD, H), jnp.float32),
)


def _update_body(x_ref, a1p_ref, r_ref, rb_ref, gam_ref, bet_ref, o_ref):
    agg = a1p_ref[0, :N] + a1p_ref[1, :N]
    t = (
        agg
        + jnp.dot(x_ref[...], r_ref[...], preferred_element_type=jnp.float32)
        + rb_ref[...]
    )
    mu = jnp.mean(t, axis=0, keepdims=True)
    var = jnp.mean((t - mu) ** 2, axis=0, keepdims=True)
    xn = (t - mu) / jnp.sqrt(var + 1e-5) * gam_ref[...] + bet_ref[...]
    o_ref[...] = jnp.maximum(xn, 0.0)


_update_call = pl.pallas_call(
    _update_body,
    out_shape=jax.ShapeDtypeStruct((N, H), jnp.float32),
)


def _pair_body(p0_ref, p1_ref, pf_ref, w1_ref, b1_ref, w2_ref, b2_ref,
               w3_ref, b3_ref, o_ref):
    c = jnp.concatenate([p0_ref[...], p1_ref[...], pf_ref[...]], axis=1)
    h1 = jnp.maximum(
        jnp.dot(c, w1_ref[...], preferred_element_type=jnp.float32) + b1_ref[...],
        0.0,
    )
    h2 = jnp.maximum(
        jnp.dot(h1, w2_ref[...], preferred_element_type=jnp.float32) + b2_ref[...],
        0.0,
    )
    o_ref[...] = (
        jnp.dot(h2, w3_ref[...], preferred_element_type=jnp.float32) + b3_ref[...]
    )


_pair_call = pl.pallas_call(
    _pair_body,
    grid=(P // BLK_P,),
    in_specs=[
        pl.BlockSpec((BLK_P, H), lambda i: (i, 0)),
        pl.BlockSpec((BLK_P, H), lambda i: (i, 0)),
        pl.BlockSpec((BLK_P, FP), lambda i: (i, 0)),
        pl.BlockSpec((2 * H + FP, 2 * H), lambda i: (0, 0)),
        pl.BlockSpec((1, 2 * H), lambda i: (0, 0)),
        pl.BlockSpec((2 * H, H), lambda i: (0, 0)),
        pl.BlockSpec((1, H), lambda i: (0, 0)),
        pl.BlockSpec((H, 1), lambda i: (0, 0)),
        pl.BlockSpec((1, 1), lambda i: (0, 0)),
    ],
    out_specs=pl.BlockSpec((BLK_P, 1), lambda i: (i, 0)),
    out_shape=jax.ShapeDtypeStruct((P, 1), jnp.float32),
)


_SC_MESH = plsc.VectorSubcoreMesh(
    core_axis_name="c", subcore_axis_name="s", num_cores=NC, num_subcores=NS
)

_SC_PARAMS = pltpu.CompilerParams(
    needs_layout_passes=False, use_tc_tiling_on_sc=False
)


def _make_gather(nch, rows):
    @functools.partial(
        pl.kernel,
        out_type=jax.ShapeDtypeStruct((NW, nch, rows, H), jnp.float32),
        mesh=_SC_MESH,
        compiler_params=_SC_PARAMS,
        scratch_types=[
            pltpu.VMEM((rows,), jnp.int32),
            pltpu.VMEM((rows, H), jnp.float32),
            pltpu.SemaphoreType.DMA,
        ],
    )
    def gather(x_hbm, idx_hbm, out_hbm, idx_v, rows_v, sem):
        cid = lax.axis_index("c")
        sid = lax.axis_index("s")
        wid = sid * NC + cid

        def chunk_body(c, carry):
            pltpu.sync_copy(idx_hbm.at[wid, c], idx_v)
            pltpu.async_copy(x_hbm.at[idx_v], rows_v, sem).wait()
            pltpu.sync_copy(rows_v, out_hbm.at[wid, c])
            return carry

        lax.fori_loop(0, nch, chunk_body, 0)

    return gather


_edge_gather = _make_gather(ECH, CHUNK)
_pair_gather = _make_gather(PCH, PROWS)


@functools.partial(
    pl.kernel,
    out_type=jax.ShapeDtypeStruct((NC, NPAD, H), jnp.float32),
    mesh=_SC_MESH,
    compiler_params=_SC_PARAMS,
    scratch_types=[
        pltpu.VMEM((CHUNK,), jnp.int32),
        pltpu.VMEM((CHUNK, H), jnp.float32),
        pltpu.VMEM_SHARED((NPAD, H), jnp.float32),
        pltpu.SemaphoreType.DMA,
    ],
)
def _edge_scatter(msg_hbm, dst_hbm, z_hbm, out_hbm, dst_v, rows_v, acc_sh, sem):
    cid = lax.axis_index("c")
    sid = lax.axis_index("s")
    wid = sid * NC + cid

    pltpu.sync_copy(z_hbm, acc_sh.at[pl.ds(sid * ROWS_PER_TILE, ROWS_PER_TILE)])
    plsc.subcore_barrier()

    def chunk_body(c, carry):
        pltpu.sync_copy(dst_hbm.at[wid, c], dst_v)
        pltpu.async_copy(msg_hbm.at[wid, c], rows_v, sem).wait()
        pltpu.sync_copy(rows_v, acc_sh.at[dst_v], add=True)
        return carry

    lax.fori_loop(0, ECH, chunk_body, 0)
    plsc.subcore_barrier()

    pltpu.sync_copy(
        acc_sh.at[pl.ds(sid * ROWS_PER_TILE, ROWS_PER_TILE)],
        out_hbm.at[cid, pl.ds(sid * ROWS_PER_TILE, ROWS_PER_TILE)],
    )


def kernel(atom_features, edge_index, edge_attr, pair_indices, pair_features,
           W_emb, b_emb, We1, be1, We2, be2, roots, root_bias, gammas, betas,
           Wp1, bp1, Wp2, bp2, Wp3, bp3):
    pad_e = E_PAD - E
    src_p = jnp.pad(edge_index[0], (0, pad_e)).reshape(NW, ECH, CHUNK)
    dst_p = jnp.pad(edge_index[1], (0, pad_e), constant_values=N).reshape(
        NW, ECH, CHUNK)
    a_p = jnp.pad(edge_attr, ((0, pad_e), (0, 0)))
    zeros_tile = jnp.zeros((ROWS_PER_TILE, H), jnp.float32)

    x = _embed_call(atom_features, W_emb, b_emb.reshape(1, H))

    for i in range(3):
        xs = _edge_gather(x, src_p).reshape(E_PAD, H)
        msg = _msg_call(a_p, xs, We1, be1.reshape(1, H), We2,
                        be2.reshape(1, H * H))
        a1p = _edge_scatter(msg.reshape(NW, ECH, CHUNK, H), dst_p, zeros_tile)
        x = _update_call(
            x, a1p, roots[i],
            root_bias[i].reshape(1, H),
            gammas[i].reshape(1, H),
            betas[i].reshape(1, H),
        )

    idx_flat = jnp.concatenate([
        pair_indices[:, 0], pair_indices[:, 1],
        jnp.zeros((P_FLAT - 2 * P,), jnp.int32),
    ]).reshape(NW, PCH, PROWS)
    rows = _pair_gather(x, idx_flat).reshape(P_FLAT, H)
    p0 = rows[:P]
    p1 = rows[P:2 * P]

    return _pair_call(
        p0, p1, pair_features,
        Wp1, bp1.reshape(1, 2 * H),
        Wp2, bp2.reshape(1, H),
        Wp3, bp3.reshape(1, 1),
    )


# speedup vs baseline: 2.3767x; 2.3586x over previous
import functools

import jax
import jax.numpy as jnp
from jax import lax
from jax.experimental import pallas as pl
from jax.experimental.pallas import tpu as pltpu
from jax.experimental.pallas import tpu_sc as plsc

N = 10000
E = 160000
P = 50000
FA = 128
FP = 16
H = 32

NC = 2
NS = 16
NW = NC * NS

ECH = 5
CHUNK = 1024
E_PAD = NW * ECH * CHUNK
NPAD = 10240
ROWS_PER_TILE = NPAD // NS

PCH = 5
PROWS = 640
P_FLAT = NW * PCH * PROWS

MBLK = 1024
BLK_P = 2000


def _embed_body(atom_ref, wemb_ref, bemb_ref, x0_ref):
    x0_ref[...] = (
        jnp.dot(atom_ref[...], wemb_ref[...], preferred_element_type=jnp.float32)
        + bemb_ref[...]
    )


_embed_call = pl.pallas_call(
    _embed_body,
    out_shape=jax.ShapeDtypeStruct((N, H), jnp.float32),
)


def _msg_body(a_ref, xs_ref, w1_ref, b1_ref, w2_ref, b2_ref, rep_ref, o_ref):
    h = jnp.maximum(a_ref[...] * w1_ref[...] + b1_ref[...], 0.0)
    we = jnp.dot(h, w2_ref[...], preferred_element_type=jnp.float32) + b2_ref[...]
    we_b = we.astype(jnp.bfloat16).astype(jnp.float32)
    xs_rep = jnp.dot(xs_ref[...], rep_ref[...], preferred_element_type=jnp.float32)
    prod = xs_rep * we_b
    s = prod[:, :512] + prod[:, 512:]
    s = s[:, :256] + s[:, 256:]
    s = s[:, :128] + s[:, 128:]
    s = s[:, :64] + s[:, 64:]
    o_ref[...] = s[:, :H] + s[:, H:]


_msg_call = pl.pallas_call(
    _msg_body,
    grid=(E_PAD // MBLK,),
    in_specs=[
        pl.BlockSpec((MBLK, 1), lambda i: (i, 0)),
        pl.BlockSpec((MBLK, H), lambda i: (i, 0)),
        pl.BlockSpec((1, H), lambda i: (0, 0)),
        pl.BlockSpec((1, H), lambda i: (0, 0)),
        pl.BlockSpec((H, H * H), lambda i: (0, 0)),
        pl.BlockSpec((1, H * H), lambda i: (0, 0)),
        pl.BlockSpec((H, H * H), lambda i: (0, 0)),
    ],
    out_specs=pl.BlockSpec((MBLK, H), lambda i: (i, 0)),
    out_shape=jax.ShapeDtypeStruct((E_PAD, H), jnp.float32),
)


def _update_body(x_ref, a1p_ref, r_ref, rb_ref, gam_ref, bet_ref, o_ref):
    agg = a1p_ref[0, :N] + a1p_ref[1, :N]
    t = (
        agg
        + jnp.dot(x_ref[...], r_ref[...], preferred_element_type=jnp.float32)
        + rb_ref[...]
    )
    mu = jnp.mean(t, axis=0, keepdims=True)
    var = jnp.mean((t - mu) ** 2, axis=0, keepdims=True)
    xn = (t - mu) / jnp.sqrt(var + 1e-5) * gam_ref[...] + bet_ref[...]
    o_ref[...] = jnp.maximum(xn, 0.0)


_update_call = pl.pallas_call(
    _update_body,
    out_shape=jax.ShapeDtypeStruct((N, H), jnp.float32),
)


def _pair_body(p0_ref, p1_ref, pf_ref, w1_ref, b1_ref, w2_ref, b2_ref,
               w3_ref, b3_ref, o_ref):
    c = jnp.concatenate([p0_ref[...], p1_ref[...], pf_ref[...]], axis=1)
    h1 = jnp.maximum(
        jnp.dot(c, w1_ref[...], preferred_element_type=jnp.float32) + b1_ref[...],
        0.0,
    )
    h2 = jnp.maximum(
        jnp.dot(h1, w2_ref[...], preferred_element_type=jnp.float32) + b2_ref[...],
        0.0,
    )
    o_ref[...] = (
        jnp.dot(h2, w3_ref[...], preferred_element_type=jnp.float32) + b3_ref[...]
    )


_pair_call = pl.pallas_call(
    _pair_body,
    grid=(P // BLK_P,),
    in_specs=[
        pl.BlockSpec((BLK_P, H), lambda i: (i, 0)),
        pl.BlockSpec((BLK_P, H), lambda i: (i, 0)),
        pl.BlockSpec((BLK_P, FP), lambda i: (i, 0)),
        pl.BlockSpec((2 * H + FP, 2 * H), lambda i: (0, 0)),
        pl.BlockSpec((1, 2 * H), lambda i: (0, 0)),
        pl.BlockSpec((2 * H, H), lambda i: (0, 0)),
        pl.BlockSpec((1, H), lambda i: (0, 0)),
        pl.BlockSpec((H, 1), lambda i: (0, 0)),
        pl.BlockSpec((1, 1), lambda i: (0, 0)),
    ],
    out_specs=pl.BlockSpec((BLK_P, 1), lambda i: (i, 0)),
    out_shape=jax.ShapeDtypeStruct((P, 1), jnp.float32),
)


_SC_MESH = plsc.VectorSubcoreMesh(
    core_axis_name="c", subcore_axis_name="s", num_cores=NC, num_subcores=NS
)

_SC_PARAMS = pltpu.CompilerParams(
    needs_layout_passes=False, use_tc_tiling_on_sc=False
)


def _make_gather(nch, rows):
    @functools.partial(
        pl.kernel,
        out_type=jax.ShapeDtypeStruct((NW, nch, rows, H), jnp.float32),
        mesh=_SC_MESH,
        compiler_params=_SC_PARAMS,
        scratch_types=[
            pltpu.VMEM((rows,), jnp.int32),
            pltpu.VMEM((rows, H), jnp.float32),
            pltpu.SemaphoreType.DMA,
        ],
    )
    def gather(x_hbm, idx_hbm, out_hbm, idx_v, rows_v, sem):
        cid = lax.axis_index("c")
        sid = lax.axis_index("s")
        wid = sid * NC + cid

        def chunk_body(c, carry):
            pltpu.sync_copy(idx_hbm.at[wid, c], idx_v)
            pltpu.async_copy(x_hbm.at[idx_v], rows_v, sem).wait()
            pltpu.sync_copy(rows_v, out_hbm.at[wid, c])
            return carry

        lax.fori_loop(0, nch, chunk_body, 0)

    return gather


_edge_gather = _make_gather(ECH, CHUNK)
_pair_gather = _make_gather(PCH, PROWS)


@functools.partial(
    pl.kernel,
    out_type=jax.ShapeDtypeStruct((NC, NPAD, H), jnp.float32),
    mesh=_SC_MESH,
    compiler_params=_SC_PARAMS,
    scratch_types=[
        pltpu.VMEM((CHUNK,), jnp.int32),
        pltpu.VMEM((CHUNK, H), jnp.float32),
        pltpu.VMEM_SHARED((NPAD, H), jnp.float32),
        pltpu.SemaphoreType.DMA,
    ],
)
def _edge_scatter(msg_hbm, dst_hbm, z_hbm, out_hbm, dst_v, rows_v, acc_sh, sem):
    cid = lax.axis_index("c")
    sid = lax.axis_index("s")
    wid = sid * NC + cid

    pltpu.sync_copy(z_hbm, acc_sh.at[pl.ds(sid * ROWS_PER_TILE, ROWS_PER_TILE)])
    plsc.subcore_barrier()

    def chunk_body(c, carry):
        pltpu.sync_copy(dst_hbm.at[wid, c], dst_v)
        pltpu.async_copy(msg_hbm.at[wid, c], rows_v, sem).wait()
        pltpu.sync_copy(rows_v, acc_sh.at[dst_v], add=True)
        return carry

    lax.fori_loop(0, ECH, chunk_body, 0)
    plsc.subcore_barrier()

    pltpu.sync_copy(
        acc_sh.at[pl.ds(sid * ROWS_PER_TILE, ROWS_PER_TILE)],
        out_hbm.at[cid, pl.ds(sid * ROWS_PER_TILE, ROWS_PER_TILE)],
    )


def kernel(atom_features, edge_index, edge_attr, pair_indices, pair_features,
           W_emb, b_emb, We1, be1, We2, be2, roots, root_bias, gammas, betas,
           Wp1, bp1, Wp2, bp2, Wp3, bp3):
    pad_e = E_PAD - E
    src_p = jnp.pad(edge_index[0], (0, pad_e)).reshape(NW, ECH, CHUNK)
    dst_p = jnp.pad(edge_index[1], (0, pad_e), constant_values=N).reshape(
        NW, ECH, CHUNK)
    a_p = jnp.pad(edge_attr, ((0, pad_e), (0, 0)))
    zeros_tile = jnp.zeros((ROWS_PER_TILE, H), jnp.float32)
    rep = (jnp.arange(H * H)[None, :] // H == jnp.arange(H)[:, None]).astype(
        jnp.float32)

    x = _embed_call(atom_features, W_emb, b_emb.reshape(1, H))

    for i in range(3):
        xs = _edge_gather(x, src_p).reshape(E_PAD, H)
        msg = _msg_call(a_p, xs, We1, be1.reshape(1, H), We2,
                        be2.reshape(1, H * H), rep)
        a1p = _edge_scatter(msg.reshape(NW, ECH, CHUNK, H), dst_p, zeros_tile)
        x = _update_call(
            x, a1p, roots[i],
            root_bias[i].reshape(1, H),
            gammas[i].reshape(1, H),
            betas[i].reshape(1, H),
        )

    idx_flat = jnp.concatenate([
        pair_indices[:, 0], pair_indices[:, 1],
        jnp.zeros((P_FLAT - 2 * P,), jnp.int32),
    ]).reshape(NW, PCH, PROWS)
    rows = _pair_gather(x, idx_flat).reshape(P_FLAT, H)
    p0 = rows[:P]
    p1 = rows[P:2 * P]

    return _pair_call(
        p0, p1, pair_features,
        Wp1, bp1.reshape(1, 2 * H),
        Wp2, bp2.reshape(1, H),
        Wp3, bp3.reshape(1, 1),
    )


# speedup vs baseline: 2.5597x; 1.0770x over previous
import functools

import jax
import jax.numpy as jnp
from jax import lax
from jax.experimental import pallas as pl
from jax.experimental.pallas import tpu as pltpu
from jax.experimental.pallas import tpu_sc as plsc

N = 10000
E = 160000
P = 50000
FA = 128
FP = 16
H = 32

NC = 2
NS = 16
NW = NC * NS

ECH = 5
CHUNK = 1024
E_PAD = NW * ECH * CHUNK
NPAD = 10240
ROWS_PER_TILE = NPAD // NS

PCH = 5
PROWS = 640
P_FLAT = NW * PCH * PROWS

MBLK = 2048
BLK_P = 2000


def _embed_body(atom_ref, wemb_ref, bemb_ref, x0_ref):
    x0_ref[...] = (
        jnp.dot(atom_ref[...], wemb_ref[...], preferred_element_type=jnp.float32)
        + bemb_ref[...]
    )


_embed_call = pl.pallas_call(
    _embed_body,
    out_shape=jax.ShapeDtypeStruct((N, H), jnp.float32),
)


def _msg_body(a_ref, xs_ref, w1_ref, b1_ref, w2_ref, b2_ref, rep_ref, o_ref):
    h = jnp.maximum(a_ref[...] * w1_ref[...] + b1_ref[...], 0.0)
    we = jnp.dot(h, w2_ref[...], preferred_element_type=jnp.float32) + b2_ref[...]
    we_b = we.astype(jnp.bfloat16).astype(jnp.float32)
    xs_rep = jnp.dot(xs_ref[...], rep_ref[...], preferred_element_type=jnp.float32)
    prod = xs_rep * we_b
    s = prod[:, :512] + prod[:, 512:]
    s = s[:, :256] + s[:, 256:]
    s = s[:, :128] + s[:, 128:]
    s = s[:, :64] + s[:, 64:]
    o_ref[...] = s[:, :H] + s[:, H:]


_msg_call = pl.pallas_call(
    _msg_body,
    grid=(E_PAD // MBLK,),
    in_specs=[
        pl.BlockSpec((MBLK, 1), lambda i: (i, 0)),
        pl.BlockSpec((MBLK, H), lambda i: (i, 0)),
        pl.BlockSpec((1, H), lambda i: (0, 0)),
        pl.BlockSpec((1, H), lambda i: (0, 0)),
        pl.BlockSpec((H, H * H), lambda i: (0, 0)),
        pl.BlockSpec((1, H * H), lambda i: (0, 0)),
        pl.BlockSpec((H, H * H), lambda i: (0, 0)),
    ],
    out_specs=pl.BlockSpec((MBLK, H), lambda i: (i, 0)),
    out_shape=jax.ShapeDtypeStruct((E_PAD, H), jnp.float32),
)


def _update_body(x_ref, a1p_ref, r_ref, rb_ref, gam_ref, bet_ref, o_ref):
    agg = a1p_ref[0, :N] + a1p_ref[1, :N]
    t = (
        agg
        + jnp.dot(x_ref[...], r_ref[...], preferred_element_type=jnp.float32)
        + rb_ref[...]
    )
    mu = jnp.mean(t, axis=0, keepdims=True)
    var = jnp.mean((t - mu) ** 2, axis=0, keepdims=True)
    xn = (t - mu) / jnp.sqrt(var + 1e-5) * gam_ref[...] + bet_ref[...]
    o_ref[...] = jnp.maximum(xn, 0.0)


_update_call = pl.pallas_call(
    _update_body,
    out_shape=jax.ShapeDtypeStruct((N, H), jnp.float32),
)


def _pair_body(p0_ref, p1_ref, pf_ref, w1_ref, b1_ref, w2_ref, b2_ref,
               w3_ref, b3_ref, o_ref):
    c = jnp.concatenate([p0_ref[...], p1_ref[...], pf_ref[...]], axis=1)
    h1 = jnp.maximum(
        jnp.dot(c, w1_ref[...], preferred_element_type=jnp.float32) + b1_ref[...],
        0.0,
    )
    h2 = jnp.maximum(
        jnp.dot(h1, w2_ref[...], preferred_element_type=jnp.float32) + b2_ref[...],
        0.0,
    )
    o_ref[...] = (
        jnp.dot(h2, w3_ref[...], preferred_element_type=jnp.float32) + b3_ref[...]
    )


_pair_call = pl.pallas_call(
    _pair_body,
    grid=(P // BLK_P,),
    in_specs=[
        pl.BlockSpec((BLK_P, H), lambda i: (i, 0)),
        pl.BlockSpec((BLK_P, H), lambda i: (i, 0)),
        pl.BlockSpec((BLK_P, FP), lambda i: (i, 0)),
        pl.BlockSpec((2 * H + FP, 2 * H), lambda i: (0, 0)),
        pl.BlockSpec((1, 2 * H), lambda i: (0, 0)),
        pl.BlockSpec((2 * H, H), lambda i: (0, 0)),
        pl.BlockSpec((1, H), lambda i: (0, 0)),
        pl.BlockSpec((H, 1), lambda i: (0, 0)),
        pl.BlockSpec((1, 1), lambda i: (0, 0)),
    ],
    out_specs=pl.BlockSpec((BLK_P, 1), lambda i: (i, 0)),
    out_shape=jax.ShapeDtypeStruct((P, 1), jnp.float32),
)


_SC_MESH = plsc.VectorSubcoreMesh(
    core_axis_name="c", subcore_axis_name="s", num_cores=NC, num_subcores=NS
)

_SC_PARAMS = pltpu.CompilerParams(
    needs_layout_passes=False, use_tc_tiling_on_sc=False
)


def _make_gather(nch, rows):
    @functools.partial(
        pl.kernel,
        out_type=jax.ShapeDtypeStruct((NW, nch, rows, H), jnp.float32),
        mesh=_SC_MESH,
        compiler_params=_SC_PARAMS,
        scratch_types=[
            pltpu.VMEM((rows,), jnp.int32),
            pltpu.VMEM((rows,), jnp.int32),
            pltpu.VMEM((rows, H), jnp.float32),
            pltpu.VMEM((rows, H), jnp.float32),
            pltpu.SemaphoreType.DMA,
            pltpu.SemaphoreType.DMA,
            pltpu.SemaphoreType.DMA,
            pltpu.SemaphoreType.DMA,
        ],
    )
    def gather(x_hbm, idx_hbm, out_hbm, i0, i1, r0, r1, g0, g1, w0, w1):
        cid = lax.axis_index("c")
        sid = lax.axis_index("s")
        wid = sid * NC + cid

        idxs, rbufs = [i0, i1], [r0, r1]
        gsem, wsem = [g0, g1], [w0, w1]
        gd = [None, None]
        wd = [None, None]

        pltpu.sync_copy(idx_hbm.at[wid, 0], idxs[0])
        gd[0] = pltpu.async_copy(x_hbm.at[idxs[0]], rbufs[0], gsem[0])
        for c in range(nch):
            b = c % 2
            nb = (c + 1) % 2
            if c + 1 < nch:
                pltpu.sync_copy(idx_hbm.at[wid, c + 1], idxs[nb])
            gd[b].wait()
            if c + 1 < nch:
                if wd[nb] is not None:
                    wd[nb].wait()
                    wd[nb] = None
                gd[nb] = pltpu.async_copy(x_hbm.at[idxs[nb]], rbufs[nb], gsem[nb])
            wd[b] = pltpu.async_copy(rbufs[b], out_hbm.at[wid, c], wsem[b])
        for b in range(2):
            if wd[b] is not None:
                wd[b].wait()

    return gather


_edge_gather = _make_gather(ECH, CHUNK)
_pair_gather = _make_gather(PCH, PROWS)


@functools.partial(
    pl.kernel,
    out_type=jax.ShapeDtypeStruct((NC, NPAD, H), jnp.float32),
    mesh=_SC_MESH,
    compiler_params=_SC_PARAMS,
    scratch_types=[
        pltpu.VMEM((CHUNK,), jnp.int32),
        pltpu.VMEM((CHUNK,), jnp.int32),
        pltpu.VMEM((CHUNK, H), jnp.float32),
        pltpu.VMEM((CHUNK, H), jnp.float32),
        pltpu.VMEM_SHARED((NPAD, H), jnp.float32),
        pltpu.SemaphoreType.DMA,
        pltpu.SemaphoreType.DMA,
    ],
)
def _edge_scatter(msg_hbm, dst_hbm, z_hbm, out_hbm, d0, d1, m0, m1, acc_sh,
                  s0, s1):
    cid = lax.axis_index("c")
    sid = lax.axis_index("s")
    wid = sid * NC + cid

    dsts, mbufs, sems = [d0, d1], [m0, m1], [s0, s1]
    md = [None, None]

    pltpu.sync_copy(dst_hbm.at[wid, 0], dsts[0])
    md[0] = pltpu.async_copy(msg_hbm.at[wid, 0], mbufs[0], sems[0])
    pltpu.sync_copy(z_hbm, acc_sh.at[pl.ds(sid * ROWS_PER_TILE, ROWS_PER_TILE)])
    plsc.subcore_barrier()

    for c in range(ECH):
        b = c % 2
        nb = (c + 1) % 2
        if c + 1 < ECH:
            pltpu.sync_copy(dst_hbm.at[wid, c + 1], dsts[nb])
            md[nb] = pltpu.async_copy(msg_hbm.at[wid, c + 1], mbufs[nb], sems[nb])
        md[b].wait()
        pltpu.sync_copy(mbufs[b], acc_sh.at[dsts[b]], add=True)

    plsc.subcore_barrier()

    pltpu.sync_copy(
        acc_sh.at[pl.ds(sid * ROWS_PER_TILE, ROWS_PER_TILE)],
        out_hbm.at[cid, pl.ds(sid * ROWS_PER_TILE, ROWS_PER_TILE)],
    )


def kernel(atom_features, edge_index, edge_attr, pair_indices, pair_features,
           W_emb, b_emb, We1, be1, We2, be2, roots, root_bias, gammas, betas,
           Wp1, bp1, Wp2, bp2, Wp3, bp3):
    pad_e = E_PAD - E
    src_p = jnp.pad(edge_index[0], (0, pad_e)).reshape(NW, ECH, CHUNK)
    dst_p = jnp.pad(edge_index[1], (0, pad_e), constant_values=N).reshape(
        NW, ECH, CHUNK)
    a_p = jnp.pad(edge_attr, ((0, pad_e), (0, 0)))
    zeros_tile = jnp.zeros((ROWS_PER_TILE, H), jnp.float32)
    rep = (jnp.arange(H * H)[None, :] // H == jnp.arange(H)[:, None]).astype(
        jnp.float32)

    x = _embed_call(atom_features, W_emb, b_emb.reshape(1, H))

    for i in range(3):
        xs = _edge_gather(x, src_p).reshape(E_PAD, H)
        msg = _msg_call(a_p, xs, We1, be1.reshape(1, H), We2,
                        be2.reshape(1, H * H), rep)
        a1p = _edge_scatter(msg.reshape(NW, ECH, CHUNK, H), dst_p, zeros_tile)
        x = _update_call(
            x, a1p, roots[i],
            root_bias[i].reshape(1, H),
            gammas[i].reshape(1, H),
            betas[i].reshape(1, H),
        )

    idx_flat = jnp.concatenate([
        pair_indices[:, 0], pair_indices[:, 1],
        jnp.zeros((P_FLAT - 2 * P,), jnp.int32),
    ]).reshape(NW, PCH, PROWS)
    rows = _pair_gather(x, idx_flat).reshape(P_FLAT, H)
    p0 = rows[:P]
    p1 = rows[P:2 * P]

    return _pair_call(
        p0, p1, pair_features,
        Wp1, bp1.reshape(1, 2 * H),
        Wp2, bp2.reshape(1, H),
        Wp3, bp3.reshape(1, 1),
    )
